# baseline (speedup 1.0000x reference)
import jax
import jax.numpy as jnp
from jax import lax
from jax.experimental import pallas as pl
from jax.experimental.pallas import tpu as pltpu
from jax.experimental.pallas import tpu_sc as plsc

VOCAB = 100002
EMBED = 128
B, L = 4096, 200
N = B * L

_INFO = plsc.get_sparse_core_info()
NC, NS = _INFO.num_cores, _INFO.num_subcores
NW = NC * NS
PER_W = N // NW
CH = 128

SEG_SHIFT = 11
SEG = 1 << SEG_SHIFT
NSEG = -(-VOCAB // SEG)
SLACK = CH
BK = PER_W + NSEG * SLACK
NKEY = NSEG * 16
N_VCHUNKS = PER_W // 16
SHARE = SEG // NS

POS_BITS = 15


def _gather_body(
    x_hbm, w_hbm, out_hbm,
    idx_v, bucket, cnts, bases, starts, shft, stage_i, stage_p, tail_idx,
    rows0, rows1, seg0, seg1,
    lsem, tsem, g0, g1, s0, s1,
):
    rows = (rows0, rows1)
    segb = (seg0, seg1)
    gsem = (g0, g1)
    ssem = (s0, s1)
    sid = lax.axis_index("s")
    wid = sid * NC + lax.axis_index("c")
    base = wid * PER_W
    pltpu.sync_copy(x_hbm.at[pl.ds(base, PER_W)], idx_v)
    iota16 = lax.iota(jnp.int32, 16)
    zeros16 = jnp.zeros((16,), jnp.int32)

    for g in range(NSEG):
        cnts[pl.ds(g * 16, 16)] = zeros16
    shft[pl.ds(0, 16)] = zeros16

    def count_step(i, carry):
        v = idx_v[pl.ds(i * 16, 16)]
        key = ((v >> SEG_SHIFT) * 16) + iota16
        c = plsc.load_gather(cnts, [key])
        plsc.store_scatter(cnts, [key], c + 1)
        return carry

    lax.fori_loop(0, N_VCHUNKS, count_step, 0)

    def base_round(g, run):
        c = cnts[pl.ds(g * 16, 16)]
        s = c
        for k in (1, 2, 4, 8):
            shft[pl.ds(16, 16)] = s
            s = s + shft[pl.ds(16 - k, 16)]
        bg = run + (s - c) + g * SLACK
        bases[pl.ds(g * 16, 16)] = bg
        starts[pl.ds(g * 16, 16)] = bg
        shft[pl.ds(16, 16)] = s
        return run + plsc.load_gather(shft, [jnp.full((16,), 31, jnp.int32)])

    lax.fori_loop(0, NSEG, base_round, zeros16)

    def fill_step(i, carry):
        v = idx_v[pl.ds(i * 16, 16)]
        key = ((v >> SEG_SHIFT) * 16) + iota16
        b = plsc.load_gather(bases, [key])
        packed = ((v & (SEG - 1)) << POS_BITS) | (iota16 + i * 16)
        plsc.store_scatter(bucket, [b], packed)
        plsc.store_scatter(bases, [key], b + 1)
        return carry

    lax.fori_loop(0, N_VCHUNKS, fill_step, 0)

    def stage(k0, k, bb):
        for j in range(CH // 16):
            w = bucket[pl.ds(k0 + k * CH + j * 16, 16)]
            stage_i.at[bb][pl.ds(j * 16, 16)] = w >> POS_BITS
            stage_p.at[bb][pl.ds(j * 16, 16)] = (
                w & ((1 << POS_BITS) - 1)
            ) + base

    def fire_g(seg_v, bb):
        pltpu.async_copy(seg_v.at[stage_i.at[bb]], rows[bb], gsem[bb])

    def wait_g(seg_v, bb):
        pltpu.make_async_copy(
            seg_v.at[stage_i.at[bb]], rows[bb], gsem[bb]
        ).wait()

    def fire_s(bb):
        pltpu.async_copy(rows[bb], out_hbm.at[stage_p.at[bb]], ssem[bb])

    def wait_s(bb):
        pltpu.make_async_copy(
            rows[bb], out_hbm.at[stage_p.at[bb]], ssem[bb]
        ).wait()

    def fire_load(s, par):
        seg_v = segb[par]
        pltpu.async_copy(
            w_hbm.at[pl.ds(s * SEG + sid * SHARE, SHARE)],
            seg_v.at[pl.ds(sid * SHARE, SHARE)],
            lsem,
        )

    def wait_load(s, par):
        seg_v = segb[par]
        pltpu.make_async_copy(
            w_hbm.at[pl.ds(s * SEG + sid * SHARE, SHARE)],
            seg_v.at[pl.ds(sid * SHARE, SHARE)],
            lsem,
        ).wait()

    LAST = NSEG - 1
    LAST_LO = LAST * SEG
    LAST_ROWS = VOCAB - LAST_LO
    LSHARE = (LAST_ROWS // NS) & ~7
    LREM_PAD = (LAST_ROWS - NS * LSHARE + 15) & ~15

    def fire_load_last():
        seg_v = segb[LAST % 2]
        pltpu.async_copy(
            w_hbm.at[pl.ds(LAST_LO + sid * LSHARE, LSHARE)],
            seg_v.at[pl.ds(sid * LSHARE, LSHARE)],
            lsem,
        )

    def stage_last_remainder():
        seg_v = segb[LAST % 2]
        @pl.when(sid == 0)
        def _():
            for j in range(LREM_PAD // 16):
                tail_idx[pl.ds(j * 16, 16)] = jnp.minimum(
                    iota16 + (LAST_LO + NS * LSHARE + j * 16), VOCAB - 1
                )
            pltpu.async_copy(
                w_hbm.at[tail_idx], rows0.at[pl.ds(0, LREM_PAD)], tsem
            ).wait()
            pltpu.sync_copy(
                rows0.at[pl.ds(0, LREM_PAD)],
                seg_v.at[pl.ds(NS * LSHARE, LREM_PAD)],
            )

    def wait_load_last():
        seg_v = segb[LAST % 2]
        pltpu.make_async_copy(
            w_hbm.at[pl.ds(LAST_LO + sid * LSHARE, LSHARE)],
            seg_v.at[pl.ds(sid * LSHARE, LSHARE)],
            lsem,
        ).wait()

    def process(s, par, nxt):
        seg_v = segb[par]
        start = starts[pl.ds(s * 16, 16)][0]
        total = (nxt - SLACK) - start

        @pl.when(total > 0)
        def _():
            pad = plsc.load_gather(bucket, [zeros16 + start])
            for j in range(SLACK // 16):
                bucket[pl.ds(start + total + j * 16, 16)] = pad
            n_ch = (total + CH - 1) >> 7

            stage(start, 0, 0)
            fire_g(seg_v, 0)

            def pair(c2, carry):
                for p2 in (0, 1):
                    k = c2 * 2 + p2

                    @pl.when(k < n_ch)
                    def _():
                        h = k + 1
                        hb = (p2 + 1) % 2

                        @pl.when(h < n_ch)
                        def _():
                            @pl.when(h >= 2)
                            def _():
                                wait_s(hb)

                            stage(start, h, hb)
                            fire_g(seg_v, hb)

                        wait_g(seg_v, p2)
                        fire_s(p2)
                return carry

            lax.fori_loop(0, (n_ch + 1) // 2, pair, 0)
            p_last = (n_ch - 1) & 1

            @pl.when((n_ch >= 2) & (p_last == 1))
            def _():
                wait_s(0)

            @pl.when((n_ch >= 2) & (p_last == 0))
            def _():
                wait_s(1)

            @pl.when(p_last == 0)
            def _():
                wait_s(0)

            @pl.when(p_last == 1)
            def _():
                wait_s(1)

    def seg_step(s, par, static_last_fire=None):
        if static_last_fire == "last":
            wait_load_last()
        else:
            wait_load(s, par)
        plsc.subcore_barrier()
        if static_last_fire == "fire_last":
            fire_load_last()
        elif static_last_fire == "last":
            stage_last_remainder()
            plsc.subcore_barrier()
        else:
            fire_load(s + 1, (par + 1) % 2)
        nxt = (
            BK
            if static_last_fire == "last"
            else starts[pl.ds((s + 1) * 16, 16)][0]
        )
        process(s, par, nxt)

    fire_load(0, 0)

    def seg_pair(s2, carry):
        seg_step(s2 * 2, 0)
        seg_step(s2 * 2 + 1, 1)
        return carry

    lax.fori_loop(0, (NSEG - 3) // 2, seg_pair, 0)
    seg_step(NSEG - 3, 0)
    seg_step(NSEG - 2, 1, "fire_last")
    seg_step(NSEG - 1, 0, "last")


_gather = pl.kernel(
    _gather_body,
    out_type=jax.ShapeDtypeStruct((N, EMBED), jnp.float32),
    mesh=plsc.VectorSubcoreMesh(core_axis_name="c", subcore_axis_name="s"),
    compiler_params=pltpu.CompilerParams(needs_layout_passes=False),
    scratch_types=[
        pltpu.VMEM((PER_W,), jnp.int32),
        pltpu.VMEM((BK,), jnp.int32),
        pltpu.VMEM((NKEY,), jnp.int32),
        pltpu.VMEM((NKEY,), jnp.int32),
        pltpu.VMEM((NKEY,), jnp.int32),
        pltpu.VMEM((32,), jnp.int32),
        pltpu.VMEM((2, CH), jnp.int32),
        pltpu.VMEM((2, CH), jnp.int32),
        pltpu.VMEM((48,), jnp.int32),
        pltpu.VMEM((CH, EMBED), jnp.float32),
        pltpu.VMEM((CH, EMBED), jnp.float32),
        pltpu.VMEM_SHARED((SEG, EMBED), jnp.float32),
        pltpu.VMEM_SHARED((SEG, EMBED), jnp.float32),
        pltpu.SemaphoreType.DMA,
        pltpu.SemaphoreType.DMA,
        pltpu.SemaphoreType.DMA,
        pltpu.SemaphoreType.DMA,
        pltpu.SemaphoreType.DMA,
        pltpu.SemaphoreType.DMA,
    ],
)


def kernel(x, target, text_inputs, W):
    out = _gather(x.reshape(-1), W)
    return out.reshape(B, L, EMBED)


# speedup vs baseline: 1.5367x; 1.5367x over previous
import functools

import jax
import jax.numpy as jnp
from jax import lax
from jax.experimental import pallas as pl
from jax.experimental.pallas import tpu as pltpu
from jax.experimental.pallas import tpu_sc as plsc

VOCAB = 100002
EMBED = 128
B, L = 4096, 200
N = B * L

_INFO = plsc.get_sparse_core_info()
NC, NS = _INFO.num_cores, _INFO.num_subcores
NW = NC * NS
PER_W = N // NW
CH = 128
N_CHUNKS = PER_W // CH
NBUF = 5
N_GROUPS = N_CHUNKS // NBUF


LAG = 3


def _gather_body(
    x_hbm, w_hbm, out_hbm, idx_v,
    r0, r1, r2, r3, r4, g0, g1, g2, g3, g4, s0, s1, s2, s3, s4,
):
    rows = (r0, r1, r2, r3, r4)
    gsem = (g0, g1, g2, g3, g4)
    ssem = (s0, s1, s2, s3, s4)
    wid = lax.axis_index("s") * NC + lax.axis_index("c")
    base = wid * PER_W
    pltpu.sync_copy(x_hbm.at[pl.ds(base, PER_W)], idx_v)

    def fire_gather(k, b):
        pltpu.async_copy(
            w_hbm.at[idx_v.at[pl.ds(k * CH, CH)]], rows[b], gsem[b]
        )

    def wait_gather(k, b):
        pltpu.make_async_copy(
            w_hbm.at[idx_v.at[pl.ds(k * CH, CH)]], rows[b], gsem[b]
        ).wait()

    def fire_store(k, b):
        pltpu.async_copy(
            rows[b], out_hbm.at[pl.ds(base + k * CH, CH)], ssem[b]
        )

    def wait_store(b):
        pltpu.make_async_copy(
            rows[b], out_hbm.at[pl.ds(base, CH)], ssem[b]
        ).wait()

    for b in range(LAG):
        fire_gather(b, b)

    def group(g, carry):
        for b in range(NBUF):
            k = g * NBUF + b
            h_b = (b + LAG) % NBUF

            @pl.when(k + LAG < N_CHUNKS)
            def _():
                @pl.when(k + LAG >= NBUF)
                def _():
                    wait_store(h_b)

                fire_gather(k + LAG, h_b)

            wait_gather(k, b)
            fire_store(k, b)
        return carry

    lax.fori_loop(0, N_GROUPS, group, 0)
    for b in range(NBUF):
        wait_store(b)


_gather = pl.kernel(
    _gather_body,
    out_type=jax.ShapeDtypeStruct((N, EMBED), jnp.float32),
    mesh=plsc.VectorSubcoreMesh(core_axis_name="c", subcore_axis_name="s"),
    scratch_types=[
        pltpu.VMEM((PER_W,), jnp.int32),
        pltpu.VMEM((CH, EMBED), jnp.float32),
        pltpu.VMEM((CH, EMBED), jnp.float32),
        pltpu.VMEM((CH, EMBED), jnp.float32),
        pltpu.VMEM((CH, EMBED), jnp.float32),
        pltpu.VMEM((CH, EMBED), jnp.float32),
        pltpu.SemaphoreType.DMA,
        pltpu.SemaphoreType.DMA,
        pltpu.SemaphoreType.DMA,
        pltpu.SemaphoreType.DMA,
        pltpu.SemaphoreType.DMA,
        pltpu.SemaphoreType.DMA,
        pltpu.SemaphoreType.DMA,
        pltpu.SemaphoreType.DMA,
        pltpu.SemaphoreType.DMA,
        pltpu.SemaphoreType.DMA,
    ],
)


def kernel(x, target, text_inputs, W):
    out = _gather(x.reshape(-1), W)
    return out.reshape(B, L, EMBED)
